# baseline (speedup 1.0000x reference)
import functools

import jax
import jax.numpy as jnp
from jax import lax
from jax.experimental import pallas as pl
from jax.experimental.pallas import tpu as pltpu
from jax.experimental.pallas import tpu_sc as plsc

MS = 512
TRIU = MS * (MS + 1) // 2
BATCH = 128
RB = 32
NBLK = MS // RB
NCH = MS // 16
NW = 32
SPB = BATCH // NW
OBLK = RB * MS


def _s(r):
    return 512 * r - (r * (r + 1)) // 2


_STARTS = []
_LENS = []
for _i in range(NBLK):
    _r0 = RB * _i
    _st = (_s(_r0) // 8) * 8
    _end = min(_s(_r0 + RB - 1) + MS, TRIU)
    _ln = -((-(_end - _st)) // 8) * 8
    _STARTS.append(_st)
    _LENS.append(_ln)
MAXSPAN = max(_LENS)


def _issue_in(in_hbm, iv, sem, b, blk):
    off = pl.multiple_of(b * TRIU + _STARTS[blk], 8)
    pltpu.async_copy(in_hbm.at[pl.ds(off, _LENS[blk])],
                     iv.at[pl.ds(0, _LENS[blk])], sem)


def _wait_in(in_hbm, iv, sem, blk):
    pltpu.make_async_copy(in_hbm.at[pl.ds(0, _LENS[blk])],
                          iv.at[pl.ds(0, _LENS[blk])], sem).wait()


def _compute_block(iv, ov, blk):
    r0 = RB * blk
    start = _STARTS[blk]

    def row_body(r, _):
        sr = 512 * r - (r * (r + 1)) // 2 - start
        ro = (r - r0) * MS
        m = r // 16
        for i in range(4):
            ci = jnp.maximum(m - 4 + i, 0)
            ov[pl.ds(ro + ci * 16, 16)] = jnp.zeros((16,), jnp.float32)
        v = iv[pl.ds(sr + m * 16, 16)]
        col = lax.iota(jnp.int32, 16) + m * 16
        ov[pl.ds(ro + m * 16, 16)] = jnp.where(col >= r, v, 0.0)

        @plsc.parallel_loop(m + 1, NCH, unroll=4)
        def _copy(c):
            ov[pl.ds(ro + c * 16, 16)] = iv[pl.ds(sr + c * 16, 16)]

        return 0

    lax.fori_loop(r0, r0 + RB, row_body, 0)


def _sc_body(in_hbm, out_hbm, iv0, iv1, ov0, ov1, si0, si1, so0, so1):
    wid = lax.axis_index("s") * 2 + lax.axis_index("c")
    ivs, ovs, sis, sos = (iv0, iv1), (ov0, ov1), (si0, si1), (so0, so1)

    for ov in (ov0, ov1):
        @plsc.parallel_loop(0, RB * NCH, unroll=4)
        def _zero(i):
            ov[pl.ds(i * 16, 16)] = jnp.zeros((16,), jnp.float32)

    _issue_in(in_hbm, iv0, si0, wid * SPB, 0)

    def sample_body(t, _):
        b = wid * SPB + t
        for blk in range(NBLK):
            p = blk % 2
            _wait_in(in_hbm, ivs[p], sis[p], blk)
            if blk < NBLK - 1:
                _issue_in(in_hbm, ivs[1 - p], sis[1 - p], b, blk + 1)
            else:
                nb = jnp.minimum(b + 1, BATCH - 1)
                _issue_in(in_hbm, ivs[1 - p], sis[1 - p], nb, 0)

            wait_out = pltpu.make_async_copy(
                ovs[p], out_hbm.at[pl.ds(0, OBLK)], sos[p])
            if blk >= 2:
                wait_out.wait()
            else:
                @pl.when(t > 0)
                def _():
                    wait_out.wait()

            _compute_block(ivs[p], ovs[p], blk)
            dst = pl.multiple_of(b * (MS * MS) + blk * OBLK, 8)
            pltpu.async_copy(ovs[p], out_hbm.at[pl.ds(dst, OBLK)], sos[p])
        return 0

    lax.fori_loop(0, SPB, sample_body, 0)

    for p in (0, 1):
        pltpu.make_async_copy(ovs[p], out_hbm.at[pl.ds(0, OBLK)],
                              sos[p]).wait()
    _wait_in(in_hbm, iv0, si0, 0)


def kernel(inputs):
    mesh = plsc.VectorSubcoreMesh(core_axis_name="c", subcore_axis_name="s")
    run = functools.partial(
        pl.kernel,
        mesh=mesh,
        out_type=jax.ShapeDtypeStruct((BATCH * MS * MS,), jnp.float32),
        scratch_types=[
            pltpu.VMEM((MAXSPAN,), jnp.float32),
            pltpu.VMEM((MAXSPAN,), jnp.float32),
            pltpu.VMEM((OBLK,), jnp.float32),
            pltpu.VMEM((OBLK,), jnp.float32),
            pltpu.SemaphoreType.DMA,
            pltpu.SemaphoreType.DMA,
            pltpu.SemaphoreType.DMA,
            pltpu.SemaphoreType.DMA,
        ],
    )

    @run
    def _k(in_hbm, out_hbm, iv0, iv1, ov0, ov1, si0, si1, so0, so1):
        _sc_body(in_hbm, out_hbm, iv0, iv1, ov0, ov1, si0, si1, so0, so1)

    return _k(inputs.reshape(-1)).reshape(BATCH, MS, MS)


# speedup vs baseline: 2.3487x; 2.3487x over previous
import functools

import jax
import jax.numpy as jnp
from jax import lax
from jax.experimental import pallas as pl
from jax.experimental.pallas import tpu as pltpu
from jax.experimental.pallas import tpu_sc as plsc

MS = 512
TRIU = MS * (MS + 1) // 2
BATCH = 128
RB = 8
NK = 32
NCH = MS // 16
GS = 8
NG = BATCH // GS
LANE_T = TRIU // 128


def _s(r):
    return 512 * r - (r * (r + 1)) // 2


_CT = []
for _k in range(NK):
    _w = 0
    for _r0 in (16 * _k, 16 * _k + RB):
        _span = _s(_r0 + RB - 1) + MS - (_s(_r0) // 128) * 128
        _w = max(_w, -(-_span // 128))
    _CT.append(_w)
CTMAX = max(_CT)


def _compute_sample(iv, ov, s, r0, c0):
    lane = lax.iota(jnp.int32, 16)
    si = lane * 0 + s

    def row(j, _):
        r = r0 + j
        sr = 512 * r - (r * (r + 1)) // 2 - c0 * 128
        m = r // 16
        ov[j, pl.ds(jnp.maximum(m - 1, 0) * 16, 16)] = jnp.zeros(
            (16,), jnp.float32)
        v = plsc.load_gather(iv, [si, sr + m * 16 + lane])
        col = lane + m * 16
        ov[j, pl.ds(m * 16, 16)] = jnp.where(col >= r, v, 0.0)

        @plsc.parallel_loop(m + 1, NCH, unroll=4)
        def _copy(c):
            ov[j, pl.ds(c * 16, 16)] = plsc.load_gather(
                iv, [si, sr + c * 16 + lane])

        return 0

    lax.fori_loop(0, RB, row, 0)


def _sc_body(in_hbm, out_hbm, iv0, iv1, ov0, ov1, si0, si1, so0, so1):
    wid = lax.axis_index("s") * 2 + lax.axis_index("c")
    g = wid // 2
    h = wid % 2
    b0 = pl.multiple_of(g * GS, 8)
    ivs, sis = (iv0, iv1), (si0, si1)
    ovs, sos = (ov0, ov1), (so0, so1)

    for ov in (ov0, ov1):
        @plsc.parallel_loop(0, RB * NCH, unroll=4)
        def _zero(i):
            ov[i // NCH, pl.ds((i % NCH) * 16, 16)] = jnp.zeros(
                (16,), jnp.float32)

    def class_r0(k):
        return 16 * k + 8 * h

    def class_c0(k):
        r0 = class_r0(k)
        sr0 = 512 * r0 - (r0 * (r0 + 1)) // 2
        return jnp.minimum(sr0 // 128, LANE_T - _CT[k])

    def issue_in(k, p):
        co = pl.multiple_of(class_c0(k) * 128, 128)
        pltpu.async_copy(
            in_hbm.at[pl.ds(b0, GS), pl.ds(co, _CT[k] * 128)],
            ivs[p].at[:, pl.ds(0, _CT[k] * 128)], sis[p])

    def wait_in(k, p):
        pltpu.make_async_copy(
            in_hbm.at[pl.ds(0, GS), pl.ds(0, _CT[k] * 128)],
            ivs[p].at[:, pl.ds(0, _CT[k] * 128)], sis[p]).wait()

    def wait_out(q):
        pltpu.make_async_copy(ovs[q], out_hbm.at[0, pl.ds(0, RB), :],
                              sos[q]).wait()

    issue_in(0, 0)
    for k in range(NK):
        p = k % 2
        wait_in(k, p)
        if k < NK - 1:
            issue_in(k + 1, 1 - p)
        r0 = pl.multiple_of(class_r0(k), 8)
        c0 = class_c0(k)

        def sample_body(sp, _):
            for q in (0, 1):
                s = 2 * sp + q
                if k == 0:
                    @pl.when(s >= 2)
                    def _():
                        wait_out(q)
                else:
                    wait_out(q)
                _compute_sample(ivs[p], ovs[q], s, r0, c0)
                pltpu.async_copy(ovs[q],
                                 out_hbm.at[b0 + s, pl.ds(r0, RB), :],
                                 sos[q])
            return 0

        lax.fori_loop(0, GS // 2, sample_body, 0)

    wait_out(0)
    wait_out(1)


def kernel(inputs):
    mesh = plsc.VectorSubcoreMesh(core_axis_name="c", subcore_axis_name="s")
    run = functools.partial(
        pl.kernel,
        mesh=mesh,
        compiler_params=pltpu.CompilerParams(needs_layout_passes=False),
        out_type=jax.ShapeDtypeStruct((BATCH, MS, MS), jnp.float32),
        scratch_types=[
            pltpu.VMEM((GS, CTMAX * 128), jnp.float32),
            pltpu.VMEM((GS, CTMAX * 128), jnp.float32),
            pltpu.VMEM((RB, MS), jnp.float32),
            pltpu.VMEM((RB, MS), jnp.float32),
            pltpu.SemaphoreType.DMA,
            pltpu.SemaphoreType.DMA,
            pltpu.SemaphoreType.DMA,
            pltpu.SemaphoreType.DMA,
        ],
    )

    @run
    def _k(in_hbm, out_hbm, iv0, iv1, ov0, ov1, si0, si1, so0, so1):
        _sc_body(in_hbm, out_hbm, iv0, iv1, ov0, ov1, si0, si1, so0, so1)

    return _k(inputs)


# speedup vs baseline: 2.3521x; 1.0015x over previous
import functools

import jax
import jax.numpy as jnp
from jax import lax
from jax.experimental import pallas as pl
from jax.experimental.pallas import tpu as pltpu
from jax.experimental.pallas import tpu_sc as plsc

MS = 512
TRIU = MS * (MS + 1) // 2
BATCH = 128
RB = 8
NK = 32
NCH = MS // 16
GS = 8
LANE_T = TRIU // 128


def _s(r):
    return 512 * r - (r * (r + 1)) // 2


_CT = []
for _k in range(NK):
    _w = 0
    for _r0 in (16 * _k, 16 * _k + RB):
        _span = _s(_r0 + RB - 1) + MS - (_s(_r0) // 128) * 128
        _w = max(_w, -(-_span // 128))
    _CT.append(_w)
CTMAX = max(_CT)


def _compute_sample(iv, ov, s, r0, c0):
    lane = lax.iota(jnp.int32, 16)
    si = lane * 0 + s

    def row(j, _):
        r = r0 + j
        sr = 512 * r - (r * (r + 1)) // 2 - c0 * 128
        m = r // 16
        ov[j, pl.ds(jnp.maximum(m - 1, 0) * 16, 16)] = jnp.zeros(
            (16,), jnp.float32)
        v = plsc.load_gather(iv, [si, sr + m * 16 + lane])
        col = lane + m * 16
        ov[j, pl.ds(m * 16, 16)] = jnp.where(col >= r, v, 0.0)

        @plsc.parallel_loop(m + 1, NCH, unroll=4)
        def _copy(c):
            ov[j, pl.ds(c * 16, 16)] = plsc.load_gather(
                iv, [si, sr + c * 16 + lane])

        return 0

    lax.fori_loop(0, RB, row, 0)


def _sc_body(in_hbm, out_hbm, iv0, iv1, ov0, ov1, si0, si1, so0, so1):
    wid = lax.axis_index("s") * 2 + lax.axis_index("c")
    g = wid // 2
    h = wid % 2
    b0 = pl.multiple_of(g * GS, 8)
    ivs, sis = (iv0, iv1), (si0, si1)
    ovs, sos = (ov0, ov1), (so0, so1)

    for ov in (ov0, ov1):
        @plsc.parallel_loop(0, RB * NCH, unroll=4)
        def _zero(i):
            ov[i // NCH, pl.ds((i % NCH) * 16, 16)] = jnp.zeros(
                (16,), jnp.float32)

    def class_r0(k):
        return 16 * k + 8 * h

    def class_c0(k):
        r0 = class_r0(k)
        sr0 = 512 * r0 - (r0 * (r0 + 1)) // 2
        return jnp.minimum(sr0 // 128, LANE_T - _CT[k])

    def issue_in(k, p):
        co = pl.multiple_of(class_c0(k) * 128, 128)
        pltpu.async_copy(
            in_hbm.at[pl.ds(b0, GS), pl.ds(co, _CT[k] * 128)],
            ivs[p].at[:, pl.ds(0, _CT[k] * 128)], sis[p])

    def wait_in(k, p):
        pltpu.make_async_copy(
            in_hbm.at[pl.ds(0, GS), pl.ds(0, _CT[k] * 128)],
            ivs[p].at[:, pl.ds(0, _CT[k] * 128)], sis[p]).wait()

    def wait_out(q):
        pltpu.make_async_copy(ovs[q], out_hbm.at[0, pl.ds(0, RB), :],
                              sos[q]).wait()

    issue_in(0, 0)
    for k in range(NK):
        p = k % 2
        wait_in(k, p)
        if k < NK - 1:
            issue_in(k + 1, 1 - p)
        r0 = pl.multiple_of(class_r0(k), 8)
        c0 = class_c0(k)

        def sample_body(sp, _):
            for q in (0, 1):
                s = 2 * sp + q
                if k == 0:
                    @pl.when(s >= 2)
                    def _():
                        wait_out(q)
                else:
                    wait_out(q)
                _compute_sample(ivs[p], ovs[q], s, r0, c0)
                pltpu.async_copy(ovs[q],
                                 out_hbm.at[b0 + s, pl.ds(r0, RB), :],
                                 sos[q])
            return 0

        lax.fori_loop(0, GS // 2, sample_body, 0)

    wait_out(0)
    wait_out(1)


def kernel(inputs):
    mesh = plsc.VectorSubcoreMesh(core_axis_name="c", subcore_axis_name="s")
    run = functools.partial(
        pl.kernel,
        mesh=mesh,
        compiler_params=pltpu.CompilerParams(needs_layout_passes=False),
        out_type=jax.ShapeDtypeStruct((BATCH, MS, MS), jnp.float32),
        scratch_types=[
            pltpu.VMEM((GS, CTMAX * 128), jnp.float32),
            pltpu.VMEM((GS, CTMAX * 128), jnp.float32),
            pltpu.VMEM((RB, MS), jnp.float32),
            pltpu.VMEM((RB, MS), jnp.float32),
            pltpu.SemaphoreType.DMA,
            pltpu.SemaphoreType.DMA,
            pltpu.SemaphoreType.DMA,
            pltpu.SemaphoreType.DMA,
        ],
    )

    @run
    def _k(in_hbm, out_hbm, iv0, iv1, ov0, ov1, si0, si1, so0, so1):
        _sc_body(in_hbm, out_hbm, iv0, iv1, ov0, ov1, si0, si1, so0, so1)

    return _k(inputs)
